# baseline (speedup 1.0000x reference)
import jax
import jax.numpy as jnp
from jax.experimental import pallas as pl
from jax.experimental.pallas import tpu as pltpu

NUM_CODES = 8192
CODE_DIM = 32
BETA = 0.25
BLK_M = 512
N_CHUNKS = 4
CHUNK = NUM_CODES // N_CHUNKS


def _rnd_bf16(x):
    return x.astype(jnp.bfloat16).astype(jnp.float32)


def _vq_block_kernel(z16_ref, zsq_ref, cb16_ref, cbcat_ref, esq_ref,
                     zq_ref, idx_ref, loss_ref):
    s = jax.lax.dot_general(z16_ref[...], cb16_ref[...],
                            (((1,), (1,)), ((), ())),
                            preferred_element_type=jnp.float32)
    zsq = zsq_ref[...]
    esq = esq_ref[...]

    acc_v = acc_m = acc_i = None
    SEG = 128
    NSEG = CHUNK // SEG
    iota_seg = jax.lax.broadcasted_iota(jnp.int32, (BLK_M, SEG), 1)
    for c in range(N_CHUNKS):
        base = c * CHUNK

        def _dseg(k):
            sl = slice(base + k * SEG, base + (k + 1) * SEG)
            return (zsq + esq[:, sl]) - 2.0 * s[:, sl]

        v = _dseg(0)
        seg_no = jnp.zeros((BLK_M, SEG), jnp.int32)
        for k in range(1, NSEG):
            vs = _dseg(k)
            lt = vs < v
            v = jnp.where(lt, vs, v)
            seg_no = jnp.where(lt, k, seg_no)
        ixg = seg_no * SEG + iota_seg
        mc = jnp.min(v, axis=1, keepdims=True)
        ic = jnp.min(jnp.where(v == mc, ixg, NUM_CODES),
                     axis=1, keepdims=True) + base
        if c == 0:
            acc_v, acc_m, acc_i = _rnd_bf16(mc), mc, ic
        else:
            lt = mc < acc_v
            acc_v = jnp.where(lt, _rnd_bf16(mc), acc_v)
            acc_m = jnp.where(lt, mc, acc_m)
            acc_i = jnp.where(lt, ic, acc_i)

    iota_n = jax.lax.broadcasted_iota(jnp.int32, (BLK_M, NUM_CODES), 1)
    onehot = (iota_n == acc_i).astype(jnp.bfloat16)
    zq2 = jnp.dot(onehot, cbcat_ref[...],
                  preferred_element_type=jnp.float32)
    zq_ref[...] = zq2[:, :CODE_DIM] + zq2[:, CODE_DIM:]
    idx_ref[...] = acc_i
    loss_ref[...] = jnp.sum(acc_m).reshape(1, 1, 1)


def kernel(z_e, codebook):
    B, N, C = z_e.shape
    z_flat = z_e.reshape(-1, C)
    R = z_flat.shape[0]
    n_blocks = R // BLK_M
    z_sq = jnp.sum(z_e ** 2, axis=2).reshape(R, 1)
    e_sq = jnp.sum(codebook ** 2, axis=1)[None, :]
    z16 = z_flat.astype(jnp.bfloat16)
    cb16 = codebook.astype(jnp.bfloat16)
    cb_lo = (codebook - cb16.astype(jnp.float32)).astype(jnp.bfloat16)
    cbcat = jnp.concatenate([cb16, cb_lo], axis=1)

    zq, idx, loss = pl.pallas_call(
        _vq_block_kernel,
        grid=(n_blocks,),
        in_specs=[
            pl.BlockSpec((BLK_M, C), lambda i: (i, 0)),
            pl.BlockSpec((BLK_M, 1), lambda i: (i, 0)),
            pl.BlockSpec((NUM_CODES, C), lambda i: (0, 0)),
            pl.BlockSpec((NUM_CODES, 2 * C), lambda i: (0, 0)),
            pl.BlockSpec((1, NUM_CODES), lambda i: (0, 0)),
        ],
        out_specs=[
            pl.BlockSpec((BLK_M, C), lambda i: (i, 0)),
            pl.BlockSpec((BLK_M, 1), lambda i: (i, 0)),
            pl.BlockSpec((1, 1, 1), lambda i: (i, 0, 0)),
        ],
        out_shape=[
            jax.ShapeDtypeStruct((R, C), jnp.float32),
            jax.ShapeDtypeStruct((R, 1), jnp.int32),
            jax.ShapeDtypeStruct((n_blocks, 1, 1), jnp.float32),
        ],
        compiler_params=pltpu.CompilerParams(
            dimension_semantics=("parallel",)),
    )(z16, z_sq, cb16, cbcat, e_sq)

    z_q = zq.reshape(B, N, C)
    indices = idx.reshape(B, N)
    mse = jnp.sum(loss) / (R * C)
    vq_loss = (1.0 + BETA) * mse / C
    return (z_q, indices, vq_loss)


# speedup vs baseline: 1.0410x; 1.0410x over previous
import jax
import jax.numpy as jnp
from jax.experimental import pallas as pl
from jax.experimental.pallas import tpu as pltpu

NUM_CODES = 8192
CODE_DIM = 32
BETA = 0.25
BLK_M = 128
N_CHUNKS = 4
CHUNK = NUM_CODES // N_CHUNKS


def _rnd_bf16(x):
    return x.astype(jnp.bfloat16).astype(jnp.float32)


def _vq_block_kernel(z16_ref, zsq_ref, cb16_ref, cbcat_ref, esq_ref,
                     zq_ref, idx_ref, loss_ref):
    s = jax.lax.dot_general(z16_ref[...], cb16_ref[...],
                            (((1,), (1,)), ((), ())),
                            preferred_element_type=jnp.float32)
    zsq = zsq_ref[...]
    esq = esq_ref[...]

    acc_v = acc_m = acc_i = None
    SEG = 128
    NSEG = CHUNK // SEG
    iota_seg = jax.lax.broadcasted_iota(jnp.int32, (BLK_M, SEG), 1)
    for c in range(N_CHUNKS):
        base = c * CHUNK

        def _dseg(k):
            sl = slice(base + k * SEG, base + (k + 1) * SEG)
            return (zsq + esq[:, sl]) - 2.0 * s[:, sl]

        v = _dseg(0)
        seg_no = jnp.zeros((BLK_M, SEG), jnp.int32)
        for k in range(1, NSEG):
            vs = _dseg(k)
            lt = vs < v
            v = jnp.where(lt, vs, v)
            seg_no = jnp.where(lt, k, seg_no)
        ixg = seg_no * SEG + iota_seg
        mc = jnp.min(v, axis=1, keepdims=True)
        ic = jnp.min(jnp.where(v == mc, ixg, NUM_CODES),
                     axis=1, keepdims=True) + base
        if c == 0:
            acc_v, acc_m, acc_i = _rnd_bf16(mc), mc, ic
        else:
            lt = mc < acc_v
            acc_v = jnp.where(lt, _rnd_bf16(mc), acc_v)
            acc_m = jnp.where(lt, mc, acc_m)
            acc_i = jnp.where(lt, ic, acc_i)

    iota_n = jax.lax.broadcasted_iota(jnp.int32, (BLK_M, NUM_CODES), 1)
    onehot = (iota_n == acc_i).astype(jnp.bfloat16)
    zq2 = jnp.dot(onehot, cbcat_ref[...],
                  preferred_element_type=jnp.float32)
    zq_ref[...] = zq2[:, :CODE_DIM] + zq2[:, CODE_DIM:]
    idx_ref[...] = acc_i
    loss_ref[...] = jnp.sum(acc_m).reshape(1, 1, 1)


def kernel(z_e, codebook):
    B, N, C = z_e.shape
    z_flat = z_e.reshape(-1, C)
    R = z_flat.shape[0]
    n_blocks = R // BLK_M
    z_sq = jnp.sum(z_e ** 2, axis=2).reshape(R, 1)
    e_sq = jnp.sum(codebook ** 2, axis=1)[None, :]
    z16 = z_flat.astype(jnp.bfloat16)
    cb16 = codebook.astype(jnp.bfloat16)
    cb_lo = (codebook - cb16.astype(jnp.float32)).astype(jnp.bfloat16)
    cbcat = jnp.concatenate([cb16, cb_lo], axis=1)

    zq, idx, loss = pl.pallas_call(
        _vq_block_kernel,
        grid=(n_blocks,),
        in_specs=[
            pl.BlockSpec((BLK_M, C), lambda i: (i, 0)),
            pl.BlockSpec((BLK_M, 1), lambda i: (i, 0)),
            pl.BlockSpec((NUM_CODES, C), lambda i: (0, 0)),
            pl.BlockSpec((NUM_CODES, 2 * C), lambda i: (0, 0)),
            pl.BlockSpec((1, NUM_CODES), lambda i: (0, 0)),
        ],
        out_specs=[
            pl.BlockSpec((BLK_M, C), lambda i: (i, 0)),
            pl.BlockSpec((BLK_M, 1), lambda i: (i, 0)),
            pl.BlockSpec((1, 1, 1), lambda i: (i, 0, 0)),
        ],
        out_shape=[
            jax.ShapeDtypeStruct((R, C), jnp.float32),
            jax.ShapeDtypeStruct((R, 1), jnp.int32),
            jax.ShapeDtypeStruct((n_blocks, 1, 1), jnp.float32),
        ],
        compiler_params=pltpu.CompilerParams(
            dimension_semantics=("parallel",)),
    )(z16, z_sq, cb16, cbcat, e_sq)

    z_q = zq.reshape(B, N, C)
    indices = idx.reshape(B, N)
    mse = jnp.sum(loss) / (R * C)
    vq_loss = (1.0 + BETA) * mse / C
    return (z_q, indices, vq_loss)


# speedup vs baseline: 1.1589x; 1.1132x over previous
import jax
import jax.numpy as jnp
from jax.experimental import pallas as pl
from jax.experimental.pallas import tpu as pltpu

NUM_CODES = 8192
CODE_DIM = 32
BETA = 0.25
BLK_M = 256
N_CHUNKS = 4
CHUNK = NUM_CODES // N_CHUNKS


def _rnd_bf16(x):
    return x.astype(jnp.bfloat16).astype(jnp.float32)


def _vq_block_kernel(z16_ref, zsq_ref, cb16_ref, cbcat_ref, esq_ref,
                     zq_ref, idx_ref, loss_ref):
    s = jax.lax.dot_general(z16_ref[...], cb16_ref[...],
                            (((1,), (1,)), ((), ())),
                            preferred_element_type=jnp.float32)
    zsq = zsq_ref[...]
    esq = esq_ref[...]

    acc_v = acc_m = acc_i = None
    SEG = 128
    NSEG = CHUNK // SEG
    iota_seg = jax.lax.broadcasted_iota(jnp.int32, (BLK_M, SEG), 1)
    for c in range(N_CHUNKS):
        base = c * CHUNK

        def _dseg(k):
            sl = slice(base + k * SEG, base + (k + 1) * SEG)
            return (zsq + esq[:, sl]) - 2.0 * s[:, sl]

        v = _dseg(0)
        seg_no = jnp.zeros((BLK_M, SEG), jnp.int32)
        for k in range(1, NSEG):
            vs = _dseg(k)
            lt = vs < v
            v = jnp.where(lt, vs, v)
            seg_no = jnp.where(lt, k, seg_no)
        ixg = seg_no * SEG + iota_seg
        mc = jnp.min(v, axis=1, keepdims=True)
        ic = jnp.min(jnp.where(v == mc, ixg, NUM_CODES),
                     axis=1, keepdims=True) + base
        if c == 0:
            acc_v, acc_m, acc_i = _rnd_bf16(mc), mc, ic
        else:
            lt = mc < acc_v
            acc_v = jnp.where(lt, _rnd_bf16(mc), acc_v)
            acc_m = jnp.where(lt, mc, acc_m)
            acc_i = jnp.where(lt, ic, acc_i)

    iota_n = jax.lax.broadcasted_iota(jnp.int32, (BLK_M, NUM_CODES), 1)
    onehot = (iota_n == acc_i).astype(jnp.bfloat16)
    zq2 = jnp.dot(onehot, cbcat_ref[...],
                  preferred_element_type=jnp.float32)
    zq_ref[...] = zq2[:, :CODE_DIM] + zq2[:, CODE_DIM:]
    idx_ref[...] = acc_i
    loss_ref[...] = jnp.sum(acc_m).reshape(1, 1, 1)


def kernel(z_e, codebook):
    B, N, C = z_e.shape
    z_flat = z_e.reshape(-1, C)
    R = z_flat.shape[0]
    n_blocks = R // BLK_M
    z_sq = jnp.sum(z_e ** 2, axis=2).reshape(R, 1)
    e_sq = jnp.sum(codebook ** 2, axis=1)[None, :]
    z16 = z_flat.astype(jnp.bfloat16)
    cb16 = codebook.astype(jnp.bfloat16)
    cb_lo = (codebook - cb16.astype(jnp.float32)).astype(jnp.bfloat16)
    cbcat = jnp.concatenate([cb16, cb_lo], axis=1)

    zq, idx, loss = pl.pallas_call(
        _vq_block_kernel,
        grid=(n_blocks,),
        in_specs=[
            pl.BlockSpec((BLK_M, C), lambda i: (i, 0)),
            pl.BlockSpec((BLK_M, 1), lambda i: (i, 0)),
            pl.BlockSpec((NUM_CODES, C), lambda i: (0, 0)),
            pl.BlockSpec((NUM_CODES, 2 * C), lambda i: (0, 0)),
            pl.BlockSpec((1, NUM_CODES), lambda i: (0, 0)),
        ],
        out_specs=[
            pl.BlockSpec((BLK_M, C), lambda i: (i, 0)),
            pl.BlockSpec((BLK_M, 1), lambda i: (i, 0)),
            pl.BlockSpec((1, 1, 1), lambda i: (i, 0, 0)),
        ],
        out_shape=[
            jax.ShapeDtypeStruct((R, C), jnp.float32),
            jax.ShapeDtypeStruct((R, 1), jnp.int32),
            jax.ShapeDtypeStruct((n_blocks, 1, 1), jnp.float32),
        ],
        compiler_params=pltpu.CompilerParams(
            dimension_semantics=("arbitrary",)),
    )(z16, z_sq, cb16, cbcat, e_sq)

    z_q = zq.reshape(B, N, C)
    indices = idx.reshape(B, N)
    mse = jnp.sum(loss) / (R * C)
    vq_loss = (1.0 + BETA) * mse / C
    return (z_q, indices, vq_loss)


# speedup vs baseline: 1.3578x; 1.1717x over previous
import functools

import jax
import jax.numpy as jnp
from jax import lax
from jax.experimental import pallas as pl
from jax.experimental.pallas import tpu as pltpu
from jax.experimental.pallas import tpu_sc as plsc

NUM_CODES = 8192
CODE_DIM = 32
BETA = 0.25
BLK_M = 256
N_CHUNKS = 4
CHUNK = NUM_CODES // N_CHUNKS
SEG = 128
NSEG = CHUNK // SEG


def _rnd_bf16(x):
    return x.astype(jnp.bfloat16).astype(jnp.float32)


def _vq_block_kernel(z16_ref, zsq_ref, cb16_ref, esq_ref,
                     idx_ref, loss_ref):
    s = jax.lax.dot_general(z16_ref[...], cb16_ref[...],
                            (((1,), (1,)), ((), ())),
                            preferred_element_type=jnp.float32)
    zsq = zsq_ref[...]
    esq = esq_ref[...]

    acc_v = acc_m = acc_i = None
    iota_seg = jax.lax.broadcasted_iota(jnp.int32, (BLK_M, SEG), 1)
    for c in range(N_CHUNKS):
        base = c * CHUNK

        def _dseg(k):
            sl = slice(base + k * SEG, base + (k + 1) * SEG)
            return (zsq + esq[:, sl]) - 2.0 * s[:, sl]

        v = _dseg(0)
        seg_no = jnp.zeros((BLK_M, SEG), jnp.int32)
        for k in range(1, NSEG):
            vs = _dseg(k)
            lt = vs < v
            v = jnp.where(lt, vs, v)
            seg_no = jnp.where(lt, k, seg_no)
        ixg = seg_no * SEG + iota_seg
        mc = jnp.min(v, axis=1, keepdims=True)
        ic = jnp.min(jnp.where(v == mc, ixg, NUM_CODES),
                     axis=1, keepdims=True) + base
        if c == 0:
            acc_v, acc_m, acc_i = _rnd_bf16(mc), mc, ic
        else:
            lt = mc < acc_v
            acc_v = jnp.where(lt, _rnd_bf16(mc), acc_v)
            acc_m = jnp.where(lt, mc, acc_m)
            acc_i = jnp.where(lt, ic, acc_i)

    idx_ref[...] = acc_i
    loss_ref[...] = jnp.sum(acc_m).reshape(1, 1, 1)


_SC_INFO = plsc.get_sparse_core_info()
_SC_NW = _SC_INFO.num_cores * _SC_INFO.num_subcores
_B_PER_W = NUM_CODES // _SC_NW
_D_PAD = 128


@functools.partial(
    pl.kernel,
    mesh=plsc.VectorSubcoreMesh(core_axis_name="c", subcore_axis_name="s"),
    out_type=jax.ShapeDtypeStruct((NUM_CODES, _D_PAD), jnp.float32),
    scratch_types=[
        pltpu.VMEM((_B_PER_W,), jnp.int32),
        pltpu.VMEM((_B_PER_W, _D_PAD), jnp.float32),
        pltpu.SemaphoreType.DMA,
    ],
)
def _sc_gather(table_hbm, idx_hbm, out_hbm, idx_v, rows_v, sem):
    wid = lax.axis_index("s") * _SC_INFO.num_cores + lax.axis_index("c")
    base = wid * _B_PER_W
    pltpu.sync_copy(idx_hbm.at[pl.ds(base, _B_PER_W)], idx_v)
    pltpu.async_copy(table_hbm.at[idx_v], rows_v, sem).wait()
    pltpu.sync_copy(rows_v, out_hbm.at[pl.ds(base, _B_PER_W)])


def kernel(z_e, codebook):
    B, N, C = z_e.shape
    z_flat = z_e.reshape(-1, C)
    R = z_flat.shape[0]
    n_blocks = R // BLK_M
    z_sq = jnp.sum(z_e ** 2, axis=2).reshape(R, 1)
    e_sq = jnp.sum(codebook ** 2, axis=1)[None, :]
    z16 = z_flat.astype(jnp.bfloat16)
    cb16 = codebook.astype(jnp.bfloat16)

    idx, loss = pl.pallas_call(
        _vq_block_kernel,
        grid=(n_blocks,),
        in_specs=[
            pl.BlockSpec((BLK_M, C), lambda i: (i, 0)),
            pl.BlockSpec((BLK_M, 1), lambda i: (i, 0)),
            pl.BlockSpec((NUM_CODES, C), lambda i: (0, 0)),
            pl.BlockSpec((1, NUM_CODES), lambda i: (0, 0)),
        ],
        out_specs=[
            pl.BlockSpec((BLK_M, 1), lambda i: (i, 0)),
            pl.BlockSpec((1, 1, 1), lambda i: (i, 0, 0)),
        ],
        out_shape=[
            jax.ShapeDtypeStruct((R, 1), jnp.int32),
            jax.ShapeDtypeStruct((n_blocks, 1, 1), jnp.float32),
        ],
        compiler_params=pltpu.CompilerParams(
            dimension_semantics=("arbitrary",)),
    )(z16, z_sq, cb16, e_sq)

    indices = idx.reshape(B, N)
    cb_pad = jnp.pad(codebook, ((0, 0), (0, _D_PAD - C)))
    z_q = _sc_gather(cb_pad, idx.reshape(R))[:, :C].reshape(B, N, C)
    mse = jnp.sum(loss) / (R * C)
    vq_loss = (1.0 + BETA) * mse / C
    return (z_q, indices, vq_loss)


# speedup vs baseline: 1.4573x; 1.0733x over previous
import functools

import jax
import jax.numpy as jnp
from jax import lax
from jax.experimental import pallas as pl
from jax.experimental.pallas import tpu as pltpu
from jax.experimental.pallas import tpu_sc as plsc

NUM_CODES = 8192
CODE_DIM = 32
BETA = 0.25
BLK_M = 512
N_CHUNKS = 4
CHUNK = NUM_CODES // N_CHUNKS
SEG = 128
NSEG = CHUNK // SEG


def _rnd_bf16(x):
    return x.astype(jnp.bfloat16).astype(jnp.float32)


def _vq_block_kernel(z16_ref, zsq_ref, cb16_ref, esq_ref,
                     idx_ref, loss_ref):
    s = jax.lax.dot_general(z16_ref[...], cb16_ref[...],
                            (((1,), (1,)), ((), ())),
                            preferred_element_type=jnp.float32)
    zsq = zsq_ref[...]
    esq = esq_ref[...]

    acc_v = acc_m = acc_i = None
    iota_seg = jax.lax.broadcasted_iota(jnp.int32, (BLK_M, SEG), 1)
    for c in range(N_CHUNKS):
        base = c * CHUNK

        def _dseg(k):
            sl = slice(base + k * SEG, base + (k + 1) * SEG)
            return (zsq + esq[:, sl]) - 2.0 * s[:, sl]

        v = _dseg(0)
        seg_no = jnp.zeros((BLK_M, SEG), jnp.int32)
        for k in range(1, NSEG):
            vs = _dseg(k)
            lt = vs < v
            v = jnp.where(lt, vs, v)
            seg_no = jnp.where(lt, k, seg_no)
        ixg = seg_no * SEG + iota_seg
        mc = jnp.min(v, axis=1, keepdims=True)
        ic = jnp.min(jnp.where(v == mc, ixg, NUM_CODES),
                     axis=1, keepdims=True) + base
        if c == 0:
            acc_v, acc_m, acc_i = _rnd_bf16(mc), mc, ic
        else:
            lt = mc < acc_v
            acc_v = jnp.where(lt, _rnd_bf16(mc), acc_v)
            acc_m = jnp.where(lt, mc, acc_m)
            acc_i = jnp.where(lt, ic, acc_i)

    idx_ref[...] = acc_i
    loss_ref[...] = jnp.sum(acc_m).reshape(1, 1, 1)


_SC_INFO = plsc.get_sparse_core_info()
_SC_NW = _SC_INFO.num_cores * _SC_INFO.num_subcores
_B_PER_W = NUM_CODES // _SC_NW
_D_PAD = 128


@functools.partial(
    pl.kernel,
    mesh=plsc.VectorSubcoreMesh(core_axis_name="c", subcore_axis_name="s"),
    out_type=jax.ShapeDtypeStruct((NUM_CODES, _D_PAD), jnp.float32),
    scratch_types=[
        pltpu.VMEM((_B_PER_W,), jnp.int32),
        pltpu.VMEM((_B_PER_W, _D_PAD), jnp.float32),
        pltpu.SemaphoreType.DMA,
    ],
)
def _sc_gather(table_hbm, idx_hbm, out_hbm, idx_v, rows_v, sem):
    wid = lax.axis_index("s") * _SC_INFO.num_cores + lax.axis_index("c")
    base = wid * _B_PER_W
    pltpu.sync_copy(idx_hbm.at[pl.ds(base, _B_PER_W)], idx_v)
    pltpu.async_copy(table_hbm.at[idx_v], rows_v, sem).wait()
    pltpu.sync_copy(rows_v, out_hbm.at[pl.ds(base, _B_PER_W)])


def kernel(z_e, codebook):
    B, N, C = z_e.shape
    z_flat = z_e.reshape(-1, C)
    R = z_flat.shape[0]
    n_blocks = R // BLK_M
    z_sq = jnp.sum(z_e ** 2, axis=2).reshape(R, 1)
    e_sq = jnp.sum(codebook ** 2, axis=1)[None, :]
    z16 = z_flat.astype(jnp.bfloat16)
    cb16 = codebook.astype(jnp.bfloat16)

    idx, loss = pl.pallas_call(
        _vq_block_kernel,
        grid=(n_blocks,),
        in_specs=[
            pl.BlockSpec((BLK_M, C), lambda i: (i, 0)),
            pl.BlockSpec((BLK_M, 1), lambda i: (i, 0)),
            pl.BlockSpec((NUM_CODES, C), lambda i: (0, 0)),
            pl.BlockSpec((1, NUM_CODES), lambda i: (0, 0)),
        ],
        out_specs=[
            pl.BlockSpec((BLK_M, 1), lambda i: (i, 0)),
            pl.BlockSpec((1, 1, 1), lambda i: (i, 0, 0)),
        ],
        out_shape=[
            jax.ShapeDtypeStruct((R, 1), jnp.int32),
            jax.ShapeDtypeStruct((n_blocks, 1, 1), jnp.float32),
        ],
        compiler_params=pltpu.CompilerParams(
            dimension_semantics=("arbitrary",)),
    )(z16, z_sq, cb16, e_sq)

    indices = idx.reshape(B, N)
    cb_pad = jnp.pad(codebook, ((0, 0), (0, _D_PAD - C)))
    z_q = _sc_gather(cb_pad, idx.reshape(R))[:, :C].reshape(B, N, C)
    mse = jnp.sum(loss) / (R * C)
    vq_loss = (1.0 + BETA) * mse / C
    return (z_q, indices, vq_loss)


# speedup vs baseline: 1.4853x; 1.0192x over previous
import functools

import jax
import jax.numpy as jnp
from jax import lax
from jax.experimental import pallas as pl
from jax.experimental.pallas import tpu as pltpu
from jax.experimental.pallas import tpu_sc as plsc

NUM_CODES = 8192
CODE_DIM = 32
BETA = 0.25
BLK_M = 1024
N_CHUNKS = 4
CHUNK = NUM_CODES // N_CHUNKS
SEG = 128
NSEG = CHUNK // SEG


def _rnd_bf16(x):
    return x.astype(jnp.bfloat16).astype(jnp.float32)


def _vq_block_kernel(z16_ref, zsq_ref, cb16_ref, esq_ref,
                     idx_ref, loss_ref):
    s = jax.lax.dot_general(z16_ref[...], cb16_ref[...],
                            (((1,), (1,)), ((), ())),
                            preferred_element_type=jnp.float32)
    zsq = zsq_ref[...]
    esq = esq_ref[...]

    acc_v = acc_m = acc_i = None
    iota_seg = jax.lax.broadcasted_iota(jnp.int32, (BLK_M, SEG), 1)
    for c in range(N_CHUNKS):
        base = c * CHUNK

        def _dseg(k):
            sl = slice(base + k * SEG, base + (k + 1) * SEG)
            return (zsq + esq[:, sl]) - 2.0 * s[:, sl]

        v = _dseg(0)
        seg_no = jnp.zeros((BLK_M, SEG), jnp.int32)
        for k in range(1, NSEG):
            vs = _dseg(k)
            lt = vs < v
            v = jnp.where(lt, vs, v)
            seg_no = jnp.where(lt, k, seg_no)
        ixg = seg_no * SEG + iota_seg
        mc = jnp.min(v, axis=1, keepdims=True)
        ic = jnp.min(jnp.where(v == mc, ixg, NUM_CODES),
                     axis=1, keepdims=True) + base
        if c == 0:
            acc_v, acc_m, acc_i = _rnd_bf16(mc), mc, ic
        else:
            lt = mc < acc_v
            acc_v = jnp.where(lt, _rnd_bf16(mc), acc_v)
            acc_m = jnp.where(lt, mc, acc_m)
            acc_i = jnp.where(lt, ic, acc_i)

    idx_ref[...] = acc_i
    loss_ref[...] = jnp.sum(acc_m).reshape(1, 1, 1)


_SC_INFO = plsc.get_sparse_core_info()
_SC_NW = _SC_INFO.num_cores * _SC_INFO.num_subcores
_B_PER_W = NUM_CODES // _SC_NW
_D_PAD = 128


@functools.partial(
    pl.kernel,
    mesh=plsc.VectorSubcoreMesh(core_axis_name="c", subcore_axis_name="s"),
    out_type=jax.ShapeDtypeStruct((NUM_CODES, _D_PAD), jnp.float32),
    scratch_types=[
        pltpu.VMEM((_B_PER_W,), jnp.int32),
        pltpu.VMEM((_B_PER_W, _D_PAD), jnp.float32),
        pltpu.SemaphoreType.DMA,
    ],
)
def _sc_gather(table_hbm, idx_hbm, out_hbm, idx_v, rows_v, sem):
    wid = lax.axis_index("s") * _SC_INFO.num_cores + lax.axis_index("c")
    base = wid * _B_PER_W
    pltpu.sync_copy(idx_hbm.at[pl.ds(base, _B_PER_W)], idx_v)
    pltpu.async_copy(table_hbm.at[idx_v], rows_v, sem).wait()
    pltpu.sync_copy(rows_v, out_hbm.at[pl.ds(base, _B_PER_W)])


def kernel(z_e, codebook):
    B, N, C = z_e.shape
    z_flat = z_e.reshape(-1, C)
    R = z_flat.shape[0]
    n_blocks = R // BLK_M
    z_sq = jnp.sum(z_e ** 2, axis=2).reshape(R, 1)
    e_sq = jnp.sum(codebook ** 2, axis=1)[None, :]
    z16 = z_flat.astype(jnp.bfloat16)
    cb16 = codebook.astype(jnp.bfloat16)

    idx, loss = pl.pallas_call(
        _vq_block_kernel,
        grid=(n_blocks,),
        in_specs=[
            pl.BlockSpec((BLK_M, C), lambda i: (i, 0)),
            pl.BlockSpec((BLK_M, 1), lambda i: (i, 0)),
            pl.BlockSpec((NUM_CODES, C), lambda i: (0, 0)),
            pl.BlockSpec((1, NUM_CODES), lambda i: (0, 0)),
        ],
        out_specs=[
            pl.BlockSpec((BLK_M, 1), lambda i: (i, 0)),
            pl.BlockSpec((1, 1, 1), lambda i: (i, 0, 0)),
        ],
        out_shape=[
            jax.ShapeDtypeStruct((R, 1), jnp.int32),
            jax.ShapeDtypeStruct((n_blocks, 1, 1), jnp.float32),
        ],
        compiler_params=pltpu.CompilerParams(
            dimension_semantics=("arbitrary",)),
    )(z16, z_sq, cb16, e_sq)

    indices = idx.reshape(B, N)
    cb_pad = jnp.pad(codebook, ((0, 0), (0, _D_PAD - C)))
    z_q = _sc_gather(cb_pad, idx.reshape(R))[:, :C].reshape(B, N, C)
    mse = jnp.sum(loss) / (R * C)
    vq_loss = (1.0 + BETA) * mse / C
    return (z_q, indices, vq_loss)


# speedup vs baseline: 1.5396x; 1.0366x over previous
import functools

import jax
import jax.numpy as jnp
from jax import lax
from jax.experimental import pallas as pl
from jax.experimental.pallas import tpu as pltpu
from jax.experimental.pallas import tpu_sc as plsc

NUM_CODES = 8192
CODE_DIM = 32
BETA = 0.25
BLK_M = 2048
N_CHUNKS = 4
CHUNK = NUM_CODES // N_CHUNKS
SEG = 128
NSEG = CHUNK // SEG


def _rnd_bf16(x):
    return x.astype(jnp.bfloat16).astype(jnp.float32)


def _vq_block_kernel(z16_ref, zsq_ref, cb16_ref, esq_ref,
                     idx_ref, loss_ref):
    s = jax.lax.dot_general(z16_ref[...], cb16_ref[...],
                            (((1,), (1,)), ((), ())),
                            preferred_element_type=jnp.float32)
    zsq = zsq_ref[...]
    esq = esq_ref[...]

    acc_v = acc_m = acc_i = None
    iota_seg = jax.lax.broadcasted_iota(jnp.int32, (BLK_M, SEG), 1)
    for c in range(N_CHUNKS):
        base = c * CHUNK

        def _dseg(k):
            sl = slice(base + k * SEG, base + (k + 1) * SEG)
            return (zsq + esq[:, sl]) - 2.0 * s[:, sl]

        v = _dseg(0)
        seg_no = jnp.zeros((BLK_M, SEG), jnp.int32)
        for k in range(1, NSEG):
            vs = _dseg(k)
            lt = vs < v
            v = jnp.where(lt, vs, v)
            seg_no = jnp.where(lt, k, seg_no)
        ixg = seg_no * SEG + iota_seg
        mc = jnp.min(v, axis=1, keepdims=True)
        ic = jnp.min(jnp.where(v == mc, ixg, NUM_CODES),
                     axis=1, keepdims=True) + base
        if c == 0:
            acc_v, acc_m, acc_i = _rnd_bf16(mc), mc, ic
        else:
            lt = mc < acc_v
            acc_v = jnp.where(lt, _rnd_bf16(mc), acc_v)
            acc_m = jnp.where(lt, mc, acc_m)
            acc_i = jnp.where(lt, ic, acc_i)

    idx_ref[...] = acc_i
    loss_ref[...] = jnp.sum(acc_m).reshape(1, 1, 1)


_SC_INFO = plsc.get_sparse_core_info()
_SC_NW = _SC_INFO.num_cores * _SC_INFO.num_subcores
_B_PER_W = NUM_CODES // _SC_NW
_D_PAD = 128


@functools.partial(
    pl.kernel,
    mesh=plsc.VectorSubcoreMesh(core_axis_name="c", subcore_axis_name="s"),
    out_type=jax.ShapeDtypeStruct((NUM_CODES, _D_PAD), jnp.float32),
    scratch_types=[
        pltpu.VMEM((_B_PER_W,), jnp.int32),
        pltpu.VMEM((_B_PER_W, _D_PAD), jnp.float32),
        pltpu.SemaphoreType.DMA,
    ],
)
def _sc_gather(table_hbm, idx_hbm, out_hbm, idx_v, rows_v, sem):
    wid = lax.axis_index("s") * _SC_INFO.num_cores + lax.axis_index("c")
    base = wid * _B_PER_W
    pltpu.sync_copy(idx_hbm.at[pl.ds(base, _B_PER_W)], idx_v)
    pltpu.async_copy(table_hbm.at[idx_v], rows_v, sem).wait()
    pltpu.sync_copy(rows_v, out_hbm.at[pl.ds(base, _B_PER_W)])


def kernel(z_e, codebook):
    B, N, C = z_e.shape
    z_flat = z_e.reshape(-1, C)
    R = z_flat.shape[0]
    n_blocks = R // BLK_M
    z_sq = jnp.sum(z_e ** 2, axis=2).reshape(R, 1)
    e_sq = jnp.sum(codebook ** 2, axis=1)[None, :]
    z16 = z_flat.astype(jnp.bfloat16)
    cb16 = codebook.astype(jnp.bfloat16)

    idx, loss = pl.pallas_call(
        _vq_block_kernel,
        grid=(n_blocks,),
        in_specs=[
            pl.BlockSpec((BLK_M, C), lambda i: (i, 0)),
            pl.BlockSpec((BLK_M, 1), lambda i: (i, 0)),
            pl.BlockSpec((NUM_CODES, C), lambda i: (0, 0)),
            pl.BlockSpec((1, NUM_CODES), lambda i: (0, 0)),
        ],
        out_specs=[
            pl.BlockSpec((BLK_M, 1), lambda i: (i, 0)),
            pl.BlockSpec((1, 1, 1), lambda i: (i, 0, 0)),
        ],
        out_shape=[
            jax.ShapeDtypeStruct((R, 1), jnp.int32),
            jax.ShapeDtypeStruct((n_blocks, 1, 1), jnp.float32),
        ],
        compiler_params=pltpu.CompilerParams(
            dimension_semantics=("arbitrary",)),
    )(z16, z_sq, cb16, e_sq)

    indices = idx.reshape(B, N)
    cb_pad = jnp.pad(codebook, ((0, 0), (0, _D_PAD - C)))
    z_q = _sc_gather(cb_pad, idx.reshape(R))[:, :C].reshape(B, N, C)
    mse = jnp.sum(loss) / (R * C)
    vq_loss = (1.0 + BETA) * mse / C
    return (z_q, indices, vq_loss)
